# baseline (speedup 1.0000x reference)
import jax
import jax.numpy as jnp
import numpy as np
from jax.experimental import pallas as pl

N = 50000
E = 800000
B = 64
IN_DIM = 5
HID = 128
OUT = 64
ETA = 1.0


def _gcn_conv(x, src, dst, W, b):
    n = x.shape[0]
    loop = jnp.arange(n, dtype=src.dtype)
    s = jnp.concatenate([src, loop])
    d = jnp.concatenate([dst, loop])
    deg = jnp.zeros((n,), x.dtype).at[d].add(1.0)
    dinv = jnp.where(deg > 0, deg ** -0.5, 0.0)
    norm = dinv[s] * dinv[d]
    xw = x @ W
    out = jnp.zeros((n, W.shape[1]), x.dtype).at[d].add(xw[s] * norm[:, None])
    return out + b


def _encode(x, src, dst, batch, W1, b1, W2, b2):
    z = jax.nn.relu(_gcn_conv(x, src, dst, W1, b1))
    z = _gcn_conv(z, src, dst, W2, b2)
    cnt = jax.ops.segment_sum(jnp.ones((x.shape[0],), z.dtype), batch, num_segments=B)
    mean = jax.ops.segment_sum(z, batch, num_segments=B) / jnp.maximum(cnt, 1.0)[:, None]
    mx = jax.ops.segment_max(z, batch, num_segments=B)
    z_g = jnp.concatenate([mean, mx], axis=1)
    return z, z_g


def _perturb(p, key):
    sigma = jnp.maximum(jnp.std(p), 1e-6)
    noise = jax.random.normal(key, p.shape, p.dtype) * sigma
    return jax.lax.stop_gradient(p + ETA * noise)


def _proj_pallas(g, P1, pb1, P2, pb2, P3, pb3):
    def body(g_ref, p1, q1, p2, q2, p3, q3, o_ref):
        h = jax.nn.relu(g_ref[...] @ p1[...] + q1[...])
        h = jax.nn.relu(h @ p2[...] + q2[...])
        o_ref[...] = h @ p3[...] + q3[...]

    return pl.pallas_call(
        body,
        out_shape=jax.ShapeDtypeStruct((g.shape[0], P3.shape[1]), g.dtype),
    )(g, P1, pb1[None, :], P2, pb2[None, :], P3, pb3[None, :])


def kernel(x, edge_index, batch, W1, b1, W2, b2, Wd, bd, P1, pb1, P2, pb2, P3, pb3):
    src = edge_index[0]
    dst = edge_index[1]
    z_node, z_g = _encode(x, src, dst, batch, W1, b1, W2, b2)
    x_hat = _gcn_conv(z_node, src, dst, Wd, bd)
    nk = jax.random.split(jax.random.key(1), 4)
    W1h = _perturb(W1, nk[0])
    b1h = _perturb(b1, nk[1])
    W2h = _perturb(W2, nk[2])
    b2h = _perturb(b2, nk[3])
    zhat_node, zhat_g = _encode(x, src, dst, batch, W1h, b1h, W2h, b2h)

    h_g = _proj_pallas(z_g, P1, pb1, P2, pb2, P3, pb3)
    hhat_g = _proj_pallas(zhat_g, P1, pb1, P2, pb2, P3, pb3)
    return (z_node, z_g, zhat_node, zhat_g, h_g, hhat_g, x_hat)


# speedup vs baseline: 15.4028x; 15.4028x over previous
import functools

import jax
import jax.numpy as jnp
import numpy as np
from jax import lax
from jax.experimental import pallas as pl
from jax.experimental.pallas import tpu as pltpu
from jax.experimental.pallas import tpu_sc as plsc

N = 50000
E = 800000
B = 64
IN_DIM = 5
HID = 128
OUT = 64
ETA = 1.0

NC = 2
NS = 16
CH = 128


BSH = 12
BSZ = 1 << BSH


def _agg_dims(n, e):
    nbkt = (n + BSZ - 1) // BSZ
    qpad = BSZ + CH
    qstripe = qpad // NS
    chunks_w = (e + NC * NS * CH - 1) // (NC * NS * CH)
    epad = chunks_w * NC * NS * CH
    cap = (chunks_w + nbkt) * CH
    return nbkt, qpad, qstripe, chunks_w, epad, cap


def _popcnt(mask):
    return jnp.sum(mask.astype(jnp.int32))


def _make_sc_partition(n, e):
    nbkt, qpad, qstripe, chunks_w, epad, cap = _agg_dims(n, e)
    mesh = plsc.VectorSubcoreMesh(core_axis_name="c", subcore_axis_name="s",
                                  num_cores=NC, num_subcores=NS)

    @functools.partial(
        pl.kernel,
        out_type=[jax.ShapeDtypeStruct((NC * NS, cap), jnp.int32),
                  jax.ShapeDtypeStruct((NC * NS, cap), jnp.int32),
                  jax.ShapeDtypeStruct((NC * NS, 2, 16), jnp.int32)],
        mesh=mesh,
        compiler_params=pltpu.CompilerParams(use_tc_tiling_on_sc=False, needs_layout_passes=False),
        scratch_types=[
            pltpu.VMEM((chunks_w, CH), jnp.int32),
            pltpu.VMEM((chunks_w, CH), jnp.int32),
            pltpu.VMEM((cap,), jnp.int32),
            pltpu.VMEM((cap,), jnp.int32),
            pltpu.VMEM((2, 16), jnp.int32),
        ],
    )
    def part(srcp, dstp, psrc, pdst, meta, part_ss, part_ds, cs_v, cd_v, mt_v):
        ci = lax.axis_index("c")
        s = lax.axis_index("s")
        w = ci * NS + s
        pltpu.sync_copy(srcp.at[pl.ds(w * chunks_w, chunks_w)], part_ss)
        pltpu.sync_copy(dstp.at[pl.ds(w * chunks_w, chunks_w)], part_ds)

        zero = jnp.zeros((), jnp.int32)

        @pl.loop(0, chunks_w, init_carry=(zero,) * nbkt)
        def counts(r, carry):
            for k in range(CH // 16):
                d16 = part_ds[r, pl.ds(k * 16, 16)]
                bkt = d16 >> BSH
                carry = tuple(carry[b] + _popcnt(bkt == b) for b in range(nbkt))
            return carry

        offc, ctc = [], []
        oc = zero
        for b in range(nbkt):
            offc.append(oc)
            nch = (counts[b] + CH - 1) // CH
            ctc.append(nch)
            oc = oc + nch

        @pl.loop(0, cap // 16)
        def _(i):
            cs_v[pl.ds(i * 16, 16)] = jnp.zeros((16,), jnp.int32)
            cd_v[pl.ds(i * 16, 16)] = jnp.full((16,), BSZ, jnp.int32)

        @pl.loop(0, chunks_w, init_carry=tuple(o * CH for o in offc))
        def _(r, cur):
            for k in range(CH // 16):
                s16 = part_ss[r, pl.ds(k * 16, 16)]
                d16 = part_ds[r, pl.ds(k * 16, 16)]
                bkt = d16 >> BSH
                lv = d16 & (BSZ - 1)
                new = []
                for b in range(nbkt):
                    m = bkt == b
                    plsc.store_compressed(cs_v.at[pl.ds(cur[b], 16)], s16, mask=m)
                    plsc.store_compressed(cd_v.at[pl.ds(cur[b], 16)], lv, mask=m)
                    new.append(cur[b] + _popcnt(m))
                cur = tuple(new)
            return cur

        lanes = lax.iota(jnp.int32, 16)
        offv = jnp.zeros((16,), jnp.int32)
        ctv = jnp.zeros((16,), jnp.int32)
        for b in range(nbkt):
            offv = jnp.where(lanes == b, offc[b], offv)
            ctv = jnp.where(lanes == b, ctc[b], ctv)
        mt_v[0, pl.ds(0, 16)] = offv
        mt_v[1, pl.ds(0, 16)] = ctv

        pltpu.sync_copy(cs_v, psrc.at[w])
        pltpu.sync_copy(cd_v, pdst.at[w])
        pltpu.sync_copy(mt_v, meta.at[w])

    return part


def _make_sc_agg(n, e, c):
    nbkt, qpad, qstripe, chunks_w, epad, cap = _agg_dims(n, e)
    nb0 = nbkt // 2
    mesh = plsc.VectorSubcoreMesh(core_axis_name="c", subcore_axis_name="s",
                                  num_cores=NC, num_subcores=NS)

    @functools.partial(
        pl.kernel,
        out_type=jax.ShapeDtypeStruct((nbkt, qpad, c), jnp.float32),
        mesh=mesh,
        compiler_params=pltpu.CompilerParams(use_tc_tiling_on_sc=False, needs_layout_passes=False),
        scratch_types=[
            pltpu.VMEM((2, 2, 16), jnp.int32),
            pltpu.VMEM((8, CH), jnp.int32),
            pltpu.VMEM((8, CH), jnp.int32),
            [pltpu.VMEM((CH, c), jnp.float32) for _ in range(4)],
            [pltpu.SemaphoreType.DMA for _ in range(8)],
            [pltpu.SemaphoreType.DMA for _ in range(8)],
            [pltpu.SemaphoreType.DMA for _ in range(4)],
            pltpu.VMEM_SHARED((qpad, c), jnp.float32),
        ],
    )
    def agg(table, psrc, pdst, meta, zeros, out,
            mt_v, si, di, rows, isems, dsems, gsems, acc):
        ci = lax.axis_index("c")
        s = lax.axis_index("s")
        nb = nb0 + ci * (nbkt - 2 * nb0)
        pltpu.sync_copy(meta.at[s], mt_v.at[0])
        pltpu.sync_copy(meta.at[s + NS], mt_v.at[1])

        lanes = lax.iota(jnp.int32, 16)

        def seg_meta(k, b):
            sel = lanes == b
            off = jnp.sum(jnp.where(sel, mt_v[k, 0, pl.ds(0, 16)], 0))
            ct = jnp.sum(jnp.where(sel, mt_v[k, 1, pl.ds(0, 16)], 0))
            return off, ct

        @pl.loop(0, nb)
        def _(r):
            b = ci * nb0 + r
            pltpu.sync_copy(zeros, acc.at[pl.ds(s * qstripe, qstripe)])
            plsc.subcore_barrier()

            for k in range(2):
                w = s + k * NS
                off, ct = seg_meta(k, b)

                def idx_load(j, slot):
                    pltpu.async_copy(
                        psrc.at[w, pl.ds((off + j) * CH, CH)], si.at[slot],
                        isems[slot])
                    pltpu.async_copy(
                        pdst.at[w, pl.ds((off + j) * CH, CH)], di.at[slot],
                        dsems[slot])

                for i in range(4):
                    @pl.when(i < ct)
                    def _():
                        idx_load(i, i)

                ngr = (ct + 4 + 7) // 8

                @pl.loop(0, ngr)
                def _(g):
                    for i8 in range(8):
                        j = g * 8 + i8
                        jm4 = j - 4

                        @pl.when((jm4 >= 0) & (jm4 < ct))
                        def _():
                            pltpu.make_async_copy(
                                table.at[si.at[0]], rows[i8 % 4],
                                gsems[i8 % 4]).wait()
                            pltpu.sync_copy(rows[i8 % 4],
                                            acc.at[di.at[(i8 + 4) % 8]],
                                            add=True)

                        @pl.when(j + 4 < ct)
                        def _():
                            idx_load(j + 4, (i8 + 4) % 8)

                        @pl.when(j < ct)
                        def _():
                            pltpu.make_async_copy(
                                psrc.at[w, pl.ds(0, CH)], si.at[i8],
                                isems[i8]).wait()
                            pltpu.make_async_copy(
                                psrc.at[w, pl.ds(0, CH)], di.at[i8],
                                dsems[i8]).wait()
                            pltpu.async_copy(table.at[si.at[i8]],
                                             rows[i8 % 4], gsems[i8 % 4])

            plsc.subcore_barrier()
            pltpu.sync_copy(acc.at[pl.ds(s * qstripe, qstripe)],
                            out.at[b, pl.ds(s * qstripe, qstripe)])
            plsc.subcore_barrier()

    return agg


def _tc1(degr, x, nblk, blk):
    n, d_in = x.shape

    def body(deg_ref, x_ref, dinv_ref, g0_ref):
        deg = deg_ref[...] + 1.0
        dv = lax.rsqrt(deg)
        dinv_ref[...] = dv
        g0 = x_ref[...] * dv
        g0_ref[...] = jnp.concatenate(
            [g0, jnp.zeros((blk, 8 - d_in), jnp.float32)], axis=1)

    return pl.pallas_call(
        body,
        grid=(nblk,),
        in_specs=[pl.BlockSpec((blk, 1), lambda i: (i, 0)),
                  pl.BlockSpec((blk, d_in), lambda i: (i, 0))],
        out_specs=[pl.BlockSpec((blk, 1), lambda i: (i, 0)),
                   pl.BlockSpec((blk, 8), lambda i: (i, 0))],
        out_shape=[jax.ShapeDtypeStruct((n, 1), jnp.float32),
                   jax.ShapeDtypeStruct((n, 8), jnp.float32)],
    )(degr, x)


def _tc2(s0, g0, dinv, W1p, b1r, W1hp, b1hr, W2, W2h, nblk, blk):
    n = s0.shape[0]
    hid, out = W2.shape

    def body(s0_ref, g0_ref, dv_ref, w1, c1, w1h, c1h, w2, w2h, gu_ref, guh_ref):
        dv = dv_ref[...]
        aggx = dv * (s0_ref[...] + g0_ref[...])
        z1 = jnp.maximum(
            jnp.dot(aggx, w1[...], preferred_element_type=jnp.float32) + c1[...], 0.0)
        z1h = jnp.maximum(
            jnp.dot(aggx, w1h[...], preferred_element_type=jnp.float32) + c1h[...], 0.0)
        gu_ref[...] = dv * jnp.dot(z1, w2[...], preferred_element_type=jnp.float32)
        guh_ref[...] = dv * jnp.dot(z1h, w2h[...], preferred_element_type=jnp.float32)

    full = lambda a, b: pl.BlockSpec((a, b), lambda i: (0, 0))
    return pl.pallas_call(
        body,
        grid=(nblk,),
        in_specs=[pl.BlockSpec((blk, 8), lambda i: (i, 0)),
                  pl.BlockSpec((blk, 8), lambda i: (i, 0)),
                  pl.BlockSpec((blk, 1), lambda i: (i, 0)),
                  full(8, hid), full(1, hid), full(8, hid), full(1, hid),
                  full(hid, out), full(hid, out)],
        out_specs=[pl.BlockSpec((blk, out), lambda i: (i, 0)),
                   pl.BlockSpec((blk, out), lambda i: (i, 0))],
        out_shape=[jax.ShapeDtypeStruct((n, out), jnp.float32),
                   jax.ShapeDtypeStruct((n, out), jnp.float32)],
    )(s0, g0, dinv, W1p, b1r, W1hp, b1hr, W2, W2h)


def _tc3(Su, Suh, gu, guh, dinv, b2r, b2hr, Wdp, nblk, blk):
    n, out = Su.shape

    def body(su_ref, suh_ref, gu_ref, guh_ref, dv_ref, c2, c2h, wd,
             z_ref, zh_ref, gd_ref):
        dv = dv_ref[...]
        z = dv * (su_ref[...] + gu_ref[...]) + c2[...]
        zh = dv * (suh_ref[...] + guh_ref[...]) + c2h[...]
        z_ref[...] = z
        zh_ref[...] = zh
        gd_ref[...] = dv * jnp.dot(z, wd[...], preferred_element_type=jnp.float32)

    full = lambda a, b: pl.BlockSpec((a, b), lambda i: (0, 0))
    return pl.pallas_call(
        body,
        grid=(nblk,),
        in_specs=[pl.BlockSpec((blk, out), lambda i: (i, 0)),
                  pl.BlockSpec((blk, out), lambda i: (i, 0)),
                  pl.BlockSpec((blk, out), lambda i: (i, 0)),
                  pl.BlockSpec((blk, out), lambda i: (i, 0)),
                  pl.BlockSpec((blk, 1), lambda i: (i, 0)),
                  full(1, out), full(1, out), full(out, 8)],
        out_specs=[pl.BlockSpec((blk, out), lambda i: (i, 0)),
                   pl.BlockSpec((blk, out), lambda i: (i, 0)),
                   pl.BlockSpec((blk, 8), lambda i: (i, 0))],
        out_shape=[jax.ShapeDtypeStruct((n, out), jnp.float32),
                   jax.ShapeDtypeStruct((n, out), jnp.float32),
                   jax.ShapeDtypeStruct((n, 8), jnp.float32)],
    )(Su, Suh, gu, guh, dinv, b2r, b2hr, Wdp)


def _tc4(Sd, gd, dinv, bdr, z_node, zh_node, batch2,
         P1, pb1r, P2, pb2r, P3, pb3r, nblk, blk, nseg, d_in):
    n, out = z_node.shape
    w = 2 * out

    def body(sd_ref, gd_ref, dv_ref, cd, z_ref, zh_ref, bt_ref,
             p1, q1, p2, q2, p3, q3,
             xh_ref, zg_ref, zhg_ref, hg_ref, hhg_ref,
             sum_acc, cnt_acc, max_acc):
        i = pl.program_id(0)

        @pl.when(i == 0)
        def _():
            sum_acc[...] = jnp.zeros((nseg, w), jnp.float32)
            cnt_acc[...] = jnp.zeros((nseg, w), jnp.float32)
            max_acc[...] = jnp.full((nseg, w), -jnp.inf, jnp.float32)

        dv = dv_ref[...]
        xh16 = dv * (sd_ref[...] + gd_ref[...])
        xh_ref[...] = xh16[:, :d_in] + cd[...]

        zcat = jnp.concatenate([z_ref[...], zh_ref[...]], axis=1)
        bt = bt_ref[...]
        oh = (lax.broadcasted_iota(jnp.int32, (blk, nseg), 1) == bt
              ).astype(jnp.float32)
        dot0 = lambda a, b: lax.dot_general(a, b, (((0,), (0,)), ((), ())),
                                            preferred_element_type=jnp.float32)
        sum_acc[...] = sum_acc[...] + dot0(oh, zcat)
        cnt_acc[...] = cnt_acc[...] + dot0(oh, jnp.ones((blk, w), jnp.float32))

        jlo = jnp.min(bt)
        jhi = jnp.max(bt)

        def seg_body(j, _):
            m = jnp.max(jnp.where(bt == j, zcat, -jnp.inf), axis=0,
                        keepdims=True)
            max_acc[pl.ds(j, 1), :] = jnp.maximum(max_acc[pl.ds(j, 1), :], m)
            return 0

        lax.fori_loop(jlo, jhi + 1, seg_body, 0)

        @pl.when(i == nblk - 1)
        def _():
            mean = sum_acc[...] / jnp.maximum(cnt_acc[...], 1.0)
            mx = max_acc[...]
            zg = jnp.concatenate([mean[:, :out], mx[:, :out]], axis=1)
            zhg = jnp.concatenate([mean[:, out:], mx[:, out:]], axis=1)
            zg_ref[...] = zg
            zhg_ref[...] = zhg

            def proj(g):
                h = jnp.maximum(jnp.dot(g, p1[...],
                                        preferred_element_type=jnp.float32)
                                + q1[...], 0.0)
                h = jnp.maximum(jnp.dot(h, p2[...],
                                        preferred_element_type=jnp.float32)
                                + q2[...], 0.0)
                return jnp.dot(h, p3[...],
                               preferred_element_type=jnp.float32) + q3[...]

            hg_ref[...] = proj(zg)
            hhg_ref[...] = proj(zhg)

    full = lambda a, b: pl.BlockSpec((a, b), lambda i: (0, 0))
    p1d, p2d, p3d = P1.shape[1], P2.shape[1], P3.shape[1]
    return pl.pallas_call(
        body,
        grid=(nblk,),
        in_specs=[pl.BlockSpec((blk, 8), lambda i: (i, 0)),
                  pl.BlockSpec((blk, 8), lambda i: (i, 0)),
                  pl.BlockSpec((blk, 1), lambda i: (i, 0)),
                  full(1, d_in),
                  pl.BlockSpec((blk, out), lambda i: (i, 0)),
                  pl.BlockSpec((blk, out), lambda i: (i, 0)),
                  pl.BlockSpec((blk, 1), lambda i: (i, 0)),
                  full(w, p1d), full(1, p1d), full(p1d, p2d), full(1, p2d),
                  full(p2d, p3d), full(1, p3d)],
        out_specs=[pl.BlockSpec((blk, d_in), lambda i: (i, 0)),
                   full(nseg, w), full(nseg, w),
                   full(nseg, p3d), full(nseg, p3d)],
        out_shape=[jax.ShapeDtypeStruct((n, d_in), jnp.float32),
                   jax.ShapeDtypeStruct((nseg, w), jnp.float32),
                   jax.ShapeDtypeStruct((nseg, w), jnp.float32),
                   jax.ShapeDtypeStruct((nseg, p3d), jnp.float32),
                   jax.ShapeDtypeStruct((nseg, p3d), jnp.float32)],
        scratch_shapes=[pltpu.VMEM((nseg, w), jnp.float32),
                        pltpu.VMEM((nseg, w), jnp.float32),
                        pltpu.VMEM((nseg, w), jnp.float32)],
    )(Sd, gd, dinv, bdr, z_node, zh_node, batch2,
      P1, pb1r, P2, pb2r, P3, pb3r)


def _perturb_like(p, key):
    sigma = jnp.maximum(jnp.std(p), 1e-6)
    return p + ETA * jax.random.normal(key, p.shape, p.dtype) * sigma


def kernel(x, edge_index, batch, W1, b1, W2, b2, Wd, bd, P1, pb1, P2, pb2, P3, pb3):
    n, d_in = x.shape
    e = edge_index.shape[1]
    nbkt, qpad, qstripe, chunks_w, epad, cap = _agg_dims(n, e)
    nblk = 25
    blk = n // nblk

    nk = jax.random.split(jax.random.key(1), 4)
    W1h = _perturb_like(W1, nk[0])
    b1h = _perturb_like(b1, nk[1])
    W2h = _perturb_like(W2, nk[2])
    b2h = _perturb_like(b2, nk[3])

    pad8 = lambda wm: jnp.concatenate(
        [wm, jnp.zeros((8 - wm.shape[0], wm.shape[1]), wm.dtype)], axis=0)
    W1p, W1hp = pad8(W1), pad8(W1h)
    Wdp = jnp.concatenate(
        [Wd, jnp.zeros((Wd.shape[0], 8 - Wd.shape[1]), Wd.dtype)], axis=1)

    src = edge_index[0]
    dst = edge_index[1]
    srcp = jnp.concatenate(
        [src, jnp.zeros((epad - e,), jnp.int32)]).reshape(epad // CH, CH)
    dstp = jnp.concatenate(
        [dst, jnp.full((epad - e,), -1, jnp.int32)]).reshape(epad // CH, CH)

    zeros8 = jnp.zeros((qstripe, 8), jnp.float32)
    zeros64 = jnp.zeros((qstripe, OUT), jnp.float32)
    ones_tab = jnp.ones((n, 8), jnp.float32)
    batch2 = batch.reshape(n, 1)
    row = lambda v: v.reshape(1, -1)

    cat_b = lambda p: p[:, :BSZ, :].reshape(nbkt * BSZ, -1)[:n]

    psrc, pdst, meta = _make_sc_partition(n, e)(srcp, dstp)

    agg8 = _make_sc_agg(n, e, 8)
    degp = agg8(ones_tab, psrc, pdst, meta, zeros8)
    degr = cat_b(degp)[:, 0:1]

    dinv, g0 = _tc1(degr, x, nblk, blk)

    s0 = cat_b(agg8(g0, psrc, pdst, meta, zeros8))

    gu, guh = _tc2(s0, g0, dinv, W1p, row(b1), W1hp, row(b1h), W2, W2h,
                   nblk, blk)

    agg64 = _make_sc_agg(n, e, OUT)
    Su = cat_b(agg64(gu, psrc, pdst, meta, zeros64))
    Suh = cat_b(agg64(guh, psrc, pdst, meta, zeros64))

    z_node, zhat_node, gd = _tc3(Su, Suh, gu, guh, dinv, row(b2), row(b2h),
                                 Wdp, nblk, blk)

    Sd = cat_b(agg8(gd, psrc, pdst, meta, zeros8))

    x_hat, z_g, zhat_g, h_g, hhat_g = _tc4(
        Sd, gd, dinv, row(bd), z_node, zhat_node, batch2,
        P1, row(pb1), P2, row(pb2), P3, row(pb3), nblk, blk, B, d_in)

    return (z_node, z_g, zhat_node, zhat_g, h_g, hhat_g, x_hat)
